# Initial kernel scaffold; baseline (speedup 1.0000x reference)
import jax
import jax.numpy as jnp
from jax.experimental import pallas as pl


def kernel(x):
    raise NotImplementedError("write your pallas kernel here")


# speedup vs baseline: 329.4911x; 329.4911x over previous
import jax
import jax.numpy as jnp
from jax.experimental import pallas as pl

_P = 5
_PR = _P // 2


def _overlap_add_body(x_ref, out_ref):
    x = x_ref[...]
    h = x.shape[1]
    row = jax.lax.broadcasted_iota(jnp.int32, (1, h, 1), 1)
    acc = jnp.zeros_like(x)
    nvalid = jnp.zeros((1, h, 1), dtype=x.dtype)
    for off in range(-_PR, _PR + 1):
        m = row + off
        ok = jnp.logical_and(m >= _PR, m <= h - 1 - _PR)
        acc = acc + jnp.where(ok, x, 0.0)
        nvalid = nvalid + ok.astype(x.dtype)
    acc = acc * jnp.array(_P, x.dtype)
    counting = nvalid * jnp.array(_P, x.dtype)
    out_ref[...] = acc / counting


def kernel(x):
    B, C, H, W = x.shape
    xf = x.reshape(B * C, H, W)
    bc_block = 24
    grid = (B * C) // bc_block
    out = pl.pallas_call(
        _overlap_add_body,
        grid=(grid,),
        in_specs=[pl.BlockSpec((bc_block, H, W), lambda i: (i, 0, 0))],
        out_specs=pl.BlockSpec((bc_block, H, W), lambda i: (i, 0, 0)),
        out_shape=jax.ShapeDtypeStruct((B * C, H, W), x.dtype),
    )(xf)
    return out.reshape(B, C, H, W)


# speedup vs baseline: 437.1974x; 1.3269x over previous
import jax
import jax.numpy as jnp
from jax.experimental import pallas as pl

_P = 5
_PR = _P // 2


def _overlap_add_body(x_ref, out_ref):
    x = x_ref[...]
    h = x.shape[1]
    row = jax.lax.broadcasted_iota(jnp.int32, (1, h, 1), 1)
    acc = jnp.zeros_like(x)
    nvalid = jnp.zeros((1, h, 1), dtype=x.dtype)
    for off in range(-_PR, _PR + 1):
        m = row + off
        ok = jnp.logical_and(m >= _PR, m <= h - 1 - _PR)
        acc = acc + jnp.where(ok, x, 0.0)
        nvalid = nvalid + ok.astype(x.dtype)
    acc = acc * jnp.array(_P, x.dtype)
    counting = nvalid * jnp.array(_P, x.dtype)
    out_ref[...] = acc / counting


def kernel(x):
    B, C, H, W = x.shape
    xf = x.reshape(B * C, H, W)
    bc_block = 48
    grid = (B * C) // bc_block
    out = pl.pallas_call(
        _overlap_add_body,
        grid=(grid,),
        in_specs=[pl.BlockSpec((bc_block, H, W), lambda i: (i, 0, 0))],
        out_specs=pl.BlockSpec((bc_block, H, W), lambda i: (i, 0, 0)),
        out_shape=jax.ShapeDtypeStruct((B * C, H, W), x.dtype),
    )(xf)
    return out.reshape(B, C, H, W)


# speedup vs baseline: 541.3272x; 1.2382x over previous
import jax
import jax.numpy as jnp
from jax.experimental import pallas as pl

_P = 5
_PR = _P // 2


def _overlap_add_body(x_ref, out_ref):
    x = x_ref[...]
    h = x.shape[1]
    row = jax.lax.broadcasted_iota(jnp.int32, (1, h, 1), 1)
    acc = jnp.zeros_like(x)
    nvalid = jnp.zeros((1, h, 1), dtype=x.dtype)
    for off in range(-_PR, _PR + 1):
        m = row + off
        ok = jnp.logical_and(m >= _PR, m <= h - 1 - _PR)
        acc = acc + jnp.where(ok, x, 0.0)
        nvalid = nvalid + ok.astype(x.dtype)
    acc = acc * jnp.array(_P, x.dtype)
    counting = nvalid * jnp.array(_P, x.dtype)
    out_ref[...] = acc / counting


def kernel(x):
    B, C, H, W = x.shape
    xf = x.reshape(B * C, H, W)
    bc_block = 96
    grid = (B * C) // bc_block
    out = pl.pallas_call(
        _overlap_add_body,
        grid=(grid,),
        in_specs=[pl.BlockSpec((bc_block, H, W), lambda i: (i, 0, 0))],
        out_specs=pl.BlockSpec((bc_block, H, W), lambda i: (i, 0, 0)),
        out_shape=jax.ShapeDtypeStruct((B * C, H, W), x.dtype),
    )(xf)
    return out.reshape(B, C, H, W)
